# baseline (speedup 1.0000x reference)
import functools

import jax
import jax.numpy as jnp
from jax import lax
from jax.experimental import pallas as pl
from jax.experimental.pallas import tpu as pltpu
from jax.experimental.pallas import tpu_sc as plsc

K = 2
DIM = 32
TEMP = 0.1
NUM_CATES = 100


def _sc_gather_fn(B, D):
    info = plsc.get_sparse_core_info()
    NC, NS = info.num_cores, info.num_subcores
    NW = NC * NS
    b_per_w = B // NW
    mesh = plsc.VectorSubcoreMesh(core_axis_name="c", subcore_axis_name="s")

    @functools.partial(
        pl.kernel,
        mesh=mesh,
        out_type=jax.ShapeDtypeStruct((B, D), jnp.float32),
        scratch_types=[
            pltpu.SMEM((b_per_w,), jnp.int32),
            pltpu.VMEM((b_per_w,), jnp.int32),
            pltpu.VMEM((b_per_w, D), jnp.float32),
            pltpu.SemaphoreType.DMA,
        ],
        compiler_params=pltpu.CompilerParams(use_tc_tiling_on_sc=False),
    )
    def gather(table_hbm, idx_hbm, out_hbm, ids_s, ids_v, rows_v, sem):
        wid = lax.axis_index("s") * NC + lax.axis_index("c")
        base = wid * b_per_w
        pltpu.sync_copy(idx_hbm.at[pl.ds(base, b_per_w)], ids_v)
        copies = []
        for j in range(b_per_w // 16):
            vec = ids_v[pl.ds(j * 16, 16)]
            for i in range(16):
                copies.append(pltpu.async_copy(
                    table_hbm.at[pl.ds(vec[i], 1)],
                    rows_v.at[pl.ds(j * 16 + i, 1)], sem))
        for c in copies:
            c.wait()
        pltpu.sync_copy(rows_v, out_hbm.at[pl.ds(base, b_per_w)])

    return gather


def _tc_body(u_ref, cid_ref, item_ref, cate_ref, gv_ref, out_ref):
    u = u_ref[...]
    cid = cid_ref[...]
    item = item_ref[...]
    cate = cate_ref[...]
    gv = gv_ref[...]

    f32 = jnp.float32
    hi = lax.Precision.HIGHEST

    cg = lax.dot_general(cate, gv, (((1,), (1,)), ((), ())),
                         preferred_element_type=f32, precision=hi)
    onehot = (cid == lax.broadcasted_iota(jnp.int32, (cid.shape[0], NUM_CATES), 1)
              ).astype(f32)
    logits = lax.dot_general(onehot, cg, (((1,), (0,)), ((), ())),
                             preferred_element_type=f32, precision=hi)
    e = jnp.exp(logits * (1.0 / TEMP))
    denom = e[:, 0:1] + e[:, 1:2]
    g0 = e[:, 0:1] / denom
    g1 = e[:, 1:2] / denom

    t = jnp.sum(item * item, axis=1, keepdims=True)
    m = jnp.concatenate(
        [item * (2.0 * g0), item * (2.0 * g1), -g0, -g1, -t], axis=1)

    s0 = jnp.sum(u[:, :DIM] * u[:, :DIM], axis=1, keepdims=True)
    s1 = jnp.sum(u[:, DIM:] * u[:, DIM:], axis=1, keepdims=True)
    ones = jnp.ones_like(s0)
    a = jnp.concatenate([u, s0, s1, ones], axis=1)

    out_ref[...] = lax.dot_general(a, m, (((1,), (1,)), ((), ())),
                                   preferred_element_type=f32, precision=hi)


def _tc_call(u, cid_col, item_table, cate_table, gate_vectors):
    B = u.shape[0]
    N = item_table.shape[0]
    return pl.pallas_call(
        _tc_body,
        out_shape=jax.ShapeDtypeStruct((B, N), jnp.float32),
    )(u, cid_col, item_table, cate_table, gate_vectors)


@jax.jit
def kernel(user_ids, cate_ids, user_table, item_table, cate_table, gate_vectors):
    B = user_ids.shape[0]
    D = user_table.shape[1]
    u = _sc_gather_fn(B, D)(user_table, user_ids.astype(jnp.int32))
    cid_col = cate_ids.astype(jnp.int32).reshape(-1, 1)
    return _tc_call(u, cid_col, item_table, cate_table, gate_vectors)


# speedup vs baseline: 13.0285x; 13.0285x over previous
import functools

import jax
import jax.numpy as jnp
from jax import lax
from jax.experimental import pallas as pl
from jax.experimental.pallas import tpu as pltpu
from jax.experimental.pallas import tpu_sc as plsc

K = 2
DIM = 32
TEMP = 0.1
NUM_CATES = 100


def _sc_gather_fn(B, D):
    info = plsc.get_sparse_core_info()
    NC, NS = info.num_cores, info.num_subcores
    NW = NC * NS
    b_per_w = B // NW
    mesh = plsc.VectorSubcoreMesh(core_axis_name="c", subcore_axis_name="s")

    @functools.partial(
        pl.kernel,
        mesh=mesh,
        out_type=jax.ShapeDtypeStruct((NW, D, b_per_w), jnp.float32),
        scratch_types=[
            pltpu.VMEM((b_per_w,), jnp.int32),
            pltpu.VMEM((D, 128), jnp.float32),
            pltpu.VMEM((D, 128), jnp.float32),
            pltpu.VMEM((D, b_per_w), jnp.float32),
            pltpu.SemaphoreType.DMA,
            pltpu.SemaphoreType.DMA,
        ],
        compiler_params=pltpu.CompilerParams(needs_layout_passes=False),
    )
    def gather(tableT_hbm, idx_hbm, out_hbm, ids_v, buf0, buf1, cols_v,
               sem0, sem1):
        wid = lax.axis_index("s") * NC + lax.axis_index("c")
        base = wid * b_per_w
        pltpu.sync_copy(idx_hbm.at[pl.ds(base, b_per_w)], ids_v)
        uids = []
        for j in range(b_per_w // 16):
            vec = ids_v[pl.ds(j * 16, 16)]
            for i in range(16):
                uids.append(vec[i])
        bufs = (buf0, buf1)
        sems = (sem0, sem1)

        def issue(k):
            off = pl.multiple_of((uids[k] // 128) * 128, 128)
            return pltpu.async_copy(
                tableT_hbm.at[:, pl.ds(off, 128)], bufs[k % 2], sems[k % 2])

        rows16 = [lax.iota(jnp.int32, 16) + (16 * kk) for kk in range(D // 16)]
        handles = [issue(0), issue(1)]
        for k in range(b_per_w):
            handles[k % 2].wait()
            cvec = jnp.full((16,), uids[k] % 128, jnp.int32)
            kvec = jnp.full((16,), k, jnp.int32)
            for kk in range(D // 16):
                vals = plsc.load_gather(bufs[k % 2], [rows16[kk], cvec])
                plsc.store_scatter(cols_v, [rows16[kk], kvec], vals)
            if k + 2 < b_per_w:
                handles[k % 2] = issue(k + 2)
        pltpu.sync_copy(cols_v, out_hbm.at[wid])

    return gather


def _tc_body(ut3_ref, cid_ref, itemT_ref, cateT_ref, gv_ref, out_ref):
    ut3 = ut3_ref[...]
    ut = jnp.concatenate([ut3[w] for w in range(ut3.shape[0])], axis=1)
    cid = cid_ref[...]
    itemT = itemT_ref[...]
    cateT = cateT_ref[...]
    gv = gv_ref[...]

    f32 = jnp.float32
    hi = lax.Precision.HIGHEST
    N = cid.shape[1]

    cgT = lax.dot_general(gv, cateT, (((1,), (0,)), ((), ())),
                          preferred_element_type=f32, precision=hi)
    onehotT = (lax.broadcasted_iota(jnp.int32, (NUM_CATES, N), 0) == cid
               ).astype(f32)
    logitsT = lax.dot_general(cgT, onehotT, (((1,), (0,)), ((), ())),
                              preferred_element_type=f32, precision=hi)
    e = jnp.exp(logitsT * (1.0 / TEMP))
    denom = e[0:1, :] + e[1:2, :]
    g0 = e[0:1, :] / denom
    g1 = e[1:2, :] / denom

    t = jnp.sum(itemT * itemT, axis=0, keepdims=True)
    mt = jnp.concatenate(
        [itemT * (2.0 * g0), itemT * (2.0 * g1), -g0, -g1, -t], axis=0)

    s0 = jnp.sum(ut[:DIM, :] * ut[:DIM, :], axis=0, keepdims=True)
    s1 = jnp.sum(ut[DIM:, :] * ut[DIM:, :], axis=0, keepdims=True)
    ones = jnp.ones_like(s0)
    at = jnp.concatenate([ut, s0, s1, ones], axis=0)

    out_ref[...] = lax.dot_general(mt, at, (((0,), (0,)), ((), ())),
                                   preferred_element_type=f32, precision=hi)


def _tc_call(ut3, cid_row, itemT, cateT, gate_vectors):
    B = ut3.shape[0] * ut3.shape[2]
    N = itemT.shape[1]
    return pl.pallas_call(
        _tc_body,
        out_shape=jax.ShapeDtypeStruct((N, B), jnp.float32),
    )(ut3, cid_row, itemT, cateT, gate_vectors)


@jax.jit
def kernel(user_ids, cate_ids, user_table, item_table, cate_table, gate_vectors):
    B = user_ids.shape[0]
    D = user_table.shape[1]
    tableT = user_table.T
    ut3 = _sc_gather_fn(B, D)(tableT, user_ids.astype(jnp.int32))
    cid_row = cate_ids.astype(jnp.int32).reshape(1, -1)
    outT = _tc_call(ut3, cid_row, item_table.T, cate_table.T, gate_vectors)
    return outT.T
